# baseline (speedup 1.0000x reference)
import dataclasses
import functools

import jax
import jax.numpy as jnp
from jax import lax
from jax.experimental import pallas as pl
from jax.experimental.pallas import tpu as pltpu
from jax.experimental.pallas import tpu_sc as plsc

_MAX_NORM = 10.0
_NW = 32
_CH = 1600
_L = 16


def _renorm_body(w_ref, out_ref):
    w = w_ref[...]
    norm = jnp.sqrt(jnp.sum(w * w, axis=1, keepdims=True))
    scale = jnp.where(norm > _MAX_NORM, _MAX_NORM / jnp.maximum(norm, 1e-12), 1.0)
    out_ref[...] = w * scale


def _renorm(w):
    return pl.pallas_call(
        _renorm_body,
        out_shape=jax.ShapeDtypeStruct(w.shape, w.dtype),
    )(w)


def _make_sc_lookup(n_rows, vocab, emb_dim, dtype):
    assert n_rows % _CH == 0
    n_chunks = n_rows // _CH
    base_n = n_chunks // _NW
    rem = n_chunks % _NW
    groups = _CH // _L
    tab_len = vocab * emb_dim

    mesh = plsc.VectorSubcoreMesh(core_axis_name="c", subcore_axis_name="s")

    cp = pltpu.CompilerParams(use_tc_tiling_on_sc=False)
    if "needs_layout_passes" in pltpu.CompilerParams.__dataclass_fields__:
        cp = dataclasses.replace(cp, needs_layout_passes=False)

    @functools.partial(
        pl.kernel,
        mesh=mesh,
        compiler_params=cp,
        out_type=jax.ShapeDtypeStruct((n_rows * emb_dim,), dtype),
        scratch_types=[
            pltpu.VMEM((tab_len,), dtype),
            pltpu.VMEM((_CH,), jnp.int32),
            pltpu.VMEM((_CH,), jnp.int32),
            pltpu.VMEM((_CH * emb_dim,), dtype),
            pltpu.VMEM((_CH * emb_dim,), dtype),
            pltpu.SemaphoreType.DMA,
            pltpu.SemaphoreType.DMA,
            pltpu.SemaphoreType.DMA,
            pltpu.SemaphoreType.DMA,
        ],
    )
    def sc_lookup(table_hbm, idx_hbm, out_hbm,
                  table_v, idx0, idx1, rows0, rows1, si0, si1, so0, so1):
        wid = lax.axis_index("s") * 2 + lax.axis_index("c")
        n = base_n + jnp.where(wid < rem, 1, 0)
        idx_bufs = (idx0, idx1)
        rows_bufs = (rows0, rows1)
        sem_idx = (si0, si1)
        sem_out = (so0, so1)

        pltpu.sync_copy(table_hbm, table_v)

        def chunk_id(t):
            return wid + _NW * t

        def start_idx(t, b):
            pltpu.make_async_copy(
                idx_hbm.at[pl.ds(chunk_id(t) * _CH, _CH)], idx_bufs[b], sem_idx[b]
            ).start()

        def gather_chunk(idx_v, rows_v):
            iota = lax.iota(jnp.int32, _L)

            @pl.loop(0, groups)
            def _(g):
                base = pl.multiple_of(g * _L, _L)
                idxvec = idx_v[pl.ds(base, _L)]
                ga_base = idxvec * emb_dim
                oa_base = (iota + base) * emb_dim
                col = iota
                for _p in range(emb_dim):
                    vals = plsc.load_gather(table_v, [ga_base + col])
                    plsc.store_scatter(rows_v, [oa_base + col], vals)
                    col = jnp.bitwise_and(col + 1, emb_dim - 1)

        for b in range(2):
            @pl.when(b < n)
            def _():
                start_idx(b, b)

        @pl.loop(0, base_n + 2, step=2)
        def _(outer):
            for b in range(2):
                t = outer + b

                @pl.when(t < n)
                def _():
                    pltpu.make_async_copy(
                        idx_hbm.at[pl.ds(chunk_id(t) * _CH, _CH)],
                        idx_bufs[b], sem_idx[b]
                    ).wait()

                    @pl.when(t >= 2)
                    def _():
                        pltpu.make_async_copy(
                            rows_bufs[b],
                            out_hbm.at[pl.ds(chunk_id(t - 2) * _CH * emb_dim,
                                             _CH * emb_dim)],
                            sem_out[b],
                        ).wait()

                    gather_chunk(idx_bufs[b], rows_bufs[b])

                    pltpu.make_async_copy(
                        rows_bufs[b],
                        out_hbm.at[pl.ds(chunk_id(t) * _CH * emb_dim,
                                         _CH * emb_dim)],
                        sem_out[b],
                    ).start()

                    @pl.when(t + 2 < n)
                    def _():
                        start_idx(t + 2, b)

        for b in range(2):
            t_b = n - 1 - ((n - 1 - b) % 2)
            pltpu.make_async_copy(
                rows_bufs[b],
                out_hbm.at[pl.ds(chunk_id(t_b) * _CH * emb_dim, _CH * emb_dim)],
                sem_out[b],
            ).wait()

    return sc_lookup


def kernel(edge_attr, bond_embedding_weight):
    vocab, emb_dim = bond_embedding_weight.shape
    n_rows = edge_attr.shape[0]
    w = jnp.reshape(_renorm(bond_embedding_weight), (vocab * emb_dim,))
    sc_lookup = _make_sc_lookup(n_rows, vocab, emb_dim,
                                bond_embedding_weight.dtype)
    flat = sc_lookup(w, edge_attr)
    return jnp.reshape(flat, (n_rows, emb_dim))


# speedup vs baseline: 1.4964x; 1.4964x over previous
import dataclasses
import functools

import jax
import jax.numpy as jnp
from jax import lax
from jax.experimental import pallas as pl
from jax.experimental.pallas import tpu as pltpu
from jax.experimental.pallas import tpu_sc as plsc

_MAX_NORM = 10.0
_NW = 32
_CH = 320
_NBUF = 2
_L = 16


def _renorm_body(w_ref, out_ref):
    w = w_ref[...]
    norm = jnp.sqrt(jnp.sum(w * w, axis=1, keepdims=True))
    scale = jnp.where(norm > _MAX_NORM, _MAX_NORM / jnp.maximum(norm, 1e-12), 1.0)
    out_ref[...] = w * scale


def _renorm(w):
    return pl.pallas_call(
        _renorm_body,
        out_shape=jax.ShapeDtypeStruct(w.shape, w.dtype),
    )(w)


def _make_sc_lookup(n_rows, vocab, emb_dim, dtype):
    assert n_rows % _CH == 0
    n_chunks = n_rows // _CH
    base_n = n_chunks // _NW
    rem = n_chunks % _NW
    groups = _CH // _L

    mesh = plsc.VectorSubcoreMesh(core_axis_name="c", subcore_axis_name="s")

    cp = pltpu.CompilerParams()
    if "needs_layout_passes" in pltpu.CompilerParams.__dataclass_fields__:
        cp = dataclasses.replace(cp, needs_layout_passes=False)

    @functools.partial(
        pl.kernel,
        mesh=mesh,
        compiler_params=cp,
        out_type=jax.ShapeDtypeStruct((n_rows, emb_dim), dtype),
        scratch_types=(
            [pltpu.VMEM((vocab, emb_dim), dtype)]
            + [pltpu.VMEM((_CH,), jnp.int32) for _ in range(_NBUF)]
            + [pltpu.VMEM((_CH, emb_dim), dtype) for _ in range(_NBUF)]
            + [pltpu.SemaphoreType.DMA for _ in range(2 * _NBUF)]
        ),
    )
    def sc_lookup(table_hbm, idx_hbm, out_hbm, table_v, *bufs):
        idx_bufs = bufs[:_NBUF]
        rows_bufs = bufs[_NBUF:2 * _NBUF]
        sem_idx = bufs[2 * _NBUF:3 * _NBUF]
        sem_out = bufs[3 * _NBUF:4 * _NBUF]

        wid = lax.axis_index("s") * 2 + lax.axis_index("c")
        n = base_n + jnp.where(wid < rem, 1, 0)

        pltpu.sync_copy(table_hbm, table_v)

        def chunk_id(t):
            return wid + _NW * t

        def start_idx(t, b):
            pltpu.make_async_copy(
                idx_hbm.at[pl.ds(chunk_id(t) * _CH, _CH)], idx_bufs[b], sem_idx[b]
            ).start()

        def gather_chunk(idx_v, rows_v):
            iota = lax.iota(jnp.int32, _L)

            @pl.loop(0, groups // 4)
            def _(g):
                base = pl.multiple_of(g * (4 * _L), _L)
                idxvecs = [idx_v[pl.ds(base + k * _L, _L)] for k in range(4)]
                oas = [iota + base + k * _L for k in range(4)]
                col = iota
                for _p in range(emb_dim):
                    for k in range(4):
                        vk = plsc.load_gather(table_v, [idxvecs[k], col])
                        plsc.store_scatter(rows_v, [oas[k], col], vk)
                    col = jnp.bitwise_and(col + 1, emb_dim - 1)

        for b in range(_NBUF):
            @pl.when(b < n)
            def _():
                start_idx(b, b)

        @pl.loop(0, base_n + _NBUF, step=_NBUF)
        def _(outer):
            for b in range(_NBUF):
                t = outer + b

                @pl.when(t < n)
                def _():
                    pltpu.make_async_copy(
                        idx_hbm.at[pl.ds(chunk_id(t) * _CH, _CH)],
                        idx_bufs[b], sem_idx[b]
                    ).wait()

                    @pl.when(t >= _NBUF)
                    def _():
                        pltpu.make_async_copy(
                            rows_bufs[b],
                            out_hbm.at[pl.ds(chunk_id(t - _NBUF) * _CH, _CH)],
                            sem_out[b],
                        ).wait()

                    gather_chunk(idx_bufs[b], rows_bufs[b])

                    pltpu.make_async_copy(
                        rows_bufs[b],
                        out_hbm.at[pl.ds(chunk_id(t) * _CH, _CH)],
                        sem_out[b],
                    ).start()

                    @pl.when(t + _NBUF < n)
                    def _():
                        start_idx(t + _NBUF, b)

        for b in range(_NBUF):
            t_b = n - 1 - ((n - 1 - b) % _NBUF)
            pltpu.make_async_copy(
                rows_bufs[b],
                out_hbm.at[pl.ds(chunk_id(t_b) * _CH, _CH)],
                sem_out[b],
            ).wait()

    return sc_lookup


def kernel(edge_attr, bond_embedding_weight):
    vocab, emb_dim = bond_embedding_weight.shape
    n_rows = edge_attr.shape[0]
    w = _renorm(bond_embedding_weight)
    sc_lookup = _make_sc_lookup(n_rows, vocab, emb_dim,
                                bond_embedding_weight.dtype)
    return sc_lookup(w, edge_attr)
